# Initial kernel scaffold; baseline (speedup 1.0000x reference)
import jax
import jax.numpy as jnp
from jax.experimental import pallas as pl


def kernel(pos, pos_flipped, fea, seed, seed_fea, Wq, bq, Wk, bk, Wv, bv, Wp1, bp1, gp1, betap1, Wp2, bp2, Wa1, ba1, ga1, betaa1, Wa2, ba2, Wls, bls, Wle, ble):
    raise NotImplementedError("write your pallas kernel here")


# speedup vs baseline: 2.1077x; 2.1077x over previous
import jax
import jax.numpy as jnp
from jax.experimental import pallas as pl

B, N, M, C_IN, DIM, K = 2, 2048, 512, 256, 256, 16
POS_H, ATTN_H = 64, 512
TN = 128
NT = N // TN
TNK = TN * K
CNT = float(B * N * K)
EPS = 1e-5
F32 = jnp.float32


def _mmT(a, b):
    return jax.lax.dot_general(a, b, (((1,), (1,)), ((), ())),
                               preferred_element_type=F32,
                               precision=jax.lax.Precision.HIGHEST)


def _mm(a, b):
    return jax.lax.dot_general(a, b, (((1,), (0,)), ((), ())),
                               preferred_element_type=F32,
                               precision=jax.lax.Precision.HIGHEST)


def _proj_body(sft_ref, feat_ref, Wls_ref, bls_ref, Wk_ref, bk_ref,
               Wv_ref, bv_ref, Wq_ref, bq_ref,
               keyf_ref, val_ref, query_ref):
    v0 = _mmT(sft_ref[0], Wls_ref[...]) + bls_ref[...]
    keyf_ref[0] = _mmT(v0, Wk_ref[...]) + bk_ref[...]
    val_ref[0] = _mmT(v0, Wv_ref[...]) + bv_ref[...]
    query_ref[0] = _mmT(feat_ref[0], Wq_ref[...]) + bq_ref[...]


def _topk_body(posf_ref, seed_ref, Wp1_ref, bp1_ref,
               idx_ref, statsp_ref):
    q = posf_ref[0]
    s = seed_ref[0]
    qn = jnp.sum(q * q, axis=1, keepdims=True)
    sn = jnp.sum(s * s, axis=1)
    d = jax.lax.dot_general(q, s, (((1,), (1,)), ((), ())),
                            preferred_element_type=F32,
                            precision=jax.lax.Precision.HIGHEST)
    d = -2.0 * d
    d = d + qn
    d = d + sn[None, :]
    colid = jax.lax.broadcasted_iota(jnp.int32, (TN, M), 1)
    cols = []
    for _ in range(K):
        mn = jnp.min(d, axis=1, keepdims=True)
        am = jnp.min(jnp.where(d == mn, colid, M), axis=1)
        cols.append(am[:, None])
        d = jnp.where(colid == am[:, None], jnp.inf, d)
    idx_t = jnp.concatenate(cols, axis=1)
    idx_ref[0] = idx_t

    oh = (idx_t[:, :, None] ==
          jax.lax.broadcasted_iota(jnp.int32, (TN, K, M), 2)
          ).astype(F32).reshape(TNK, M)
    cg = _mm(oh, s)
    qrep = jnp.broadcast_to(q[:, None, :], (TN, K, 3)).reshape(TNK, 3)
    pos_rel = qrep - cg
    dis = jnp.sqrt(jnp.sum(pos_rel * pos_rel, axis=1, keepdims=True))
    h = jnp.concatenate([dis, pos_rel], axis=1)
    tp = _mmT(h, Wp1_ref[...]) + bp1_ref[...]

    first = jnp.logical_and(pl.program_id(0) == 0, pl.program_id(1) == 0)

    @pl.when(first)
    def _():
        statsp_ref[...] = jnp.zeros_like(statsp_ref)

    statsp_ref[0:1, :] += jnp.sum(tp, axis=0, keepdims=True)
    statsp_ref[1:2, :] += jnp.sum(tp * tp, axis=0, keepdims=True)


def _pass2_body(idx_ref, posf_ref, seed_ref, keyf_ref, val_ref, query_ref,
                statsp_ref, Wp1_ref, bp1_ref, gp1_ref, betap1_ref,
                Wp2_ref, bp2_ref, Wa1_ref, ba1_ref,
                ta_ref, vpe_ref, statsa_ref):
    idx_t = idx_ref[0]
    q = posf_ref[0]
    s = seed_ref[0]
    oh = (idx_t[:, :, None] ==
          jax.lax.broadcasted_iota(jnp.int32, (TN, K, M), 2)
          ).astype(F32).reshape(TNK, M)
    cg = _mm(oh, s)
    qrep = jnp.broadcast_to(q[:, None, :], (TN, K, 3)).reshape(TNK, 3)
    pos_rel = qrep - cg
    dis = jnp.sqrt(jnp.sum(pos_rel * pos_rel, axis=1, keepdims=True))
    h = jnp.concatenate([dis, pos_rel], axis=1)
    tp = _mmT(h, Wp1_ref[...]) + bp1_ref[...]

    mp = statsp_ref[0:1, :] / CNT
    vp = statsp_ref[1:2, :] / CNT - mp * mp
    xn = (tp - mp) / jnp.sqrt(vp + EPS) * gp1_ref[...] + betap1_ref[...]
    pe = _mmT(jnp.maximum(xn, 0.0), Wp2_ref[...]) + bp2_ref[...]

    key_g = _mm(oh, keyf_ref[0])
    val_g = _mm(oh, val_ref[0])
    qf = query_ref[0]
    qfrep = jnp.broadcast_to(qf[:, None, :], (TN, K, DIM)).reshape(TNK, DIM)
    x = qfrep - key_g + pe
    ta = _mmT(x, Wa1_ref[...]) + ba1_ref[...]
    ta_ref[0] = ta
    vpe_ref[0] = val_g + pe

    first = jnp.logical_and(pl.program_id(0) == 0, pl.program_id(1) == 0)

    @pl.when(first)
    def _():
        statsa_ref[...] = jnp.zeros_like(statsa_ref)

    statsa_ref[0:1, :] += jnp.sum(ta, axis=0, keepdims=True)
    statsa_ref[1:2, :] += jnp.sum(ta * ta, axis=0, keepdims=True)


def _pass3_body(ta_ref, vpe_ref, feat_ref, statsa_ref,
                ga1_ref, betaa1_ref, Wa2_ref, ba2_ref, Wle_ref, ble_ref,
                out_ref):
    ta = ta_ref[0]
    ma = statsa_ref[0:1, :] / CNT
    va = statsa_ref[1:2, :] / CNT - ma * ma
    u = (ta - ma) / jnp.sqrt(va + EPS) * ga1_ref[...] + betaa1_ref[...]
    u = jnp.maximum(u, 0.0)
    w = _mmT(u, Wa2_ref[...]) + ba2_ref[...]
    w3 = w.reshape(TN, K, DIM)
    wmax = jnp.max(w3, axis=1, keepdims=True)
    e = jnp.exp(w3 - wmax)
    sm = e / jnp.sum(e, axis=1, keepdims=True)
    agg = jnp.sum(sm * vpe_ref[0].reshape(TN, K, DIM), axis=1)
    out_ref[0] = _mmT(agg, Wle_ref[...]) + ble_ref[...] + feat_ref[0]


def kernel(pos, pos_flipped, fea, seed, seed_fea, Wq, bq, Wk, bk, Wv, bv,
           Wp1, bp1, gp1, betap1, Wp2, bp2, Wa1, ba1, ga1, betaa1, Wa2, ba2,
           Wls, bls, Wle, ble):
    del pos
    feat = jnp.transpose(fea, (0, 2, 1))
    sft = jnp.transpose(seed_fea, (0, 2, 1))
    b2 = lambda v: v.reshape(1, -1)

    keyf, val, query = pl.pallas_call(
        _proj_body,
        grid=(B,),
        in_specs=[
            pl.BlockSpec((1, M, C_IN), lambda b: (b, 0, 0)),
            pl.BlockSpec((1, N, C_IN), lambda b: (b, 0, 0)),
        ] + [pl.BlockSpec(w.shape, lambda b: tuple(0 for _ in w.shape))
             for w in (Wls, b2(bls), Wk, b2(bk), Wv, b2(bv), Wq, b2(bq))],
        out_specs=[
            pl.BlockSpec((1, M, DIM), lambda b: (b, 0, 0)),
            pl.BlockSpec((1, M, DIM), lambda b: (b, 0, 0)),
            pl.BlockSpec((1, N, DIM), lambda b: (b, 0, 0)),
        ],
        out_shape=[
            jax.ShapeDtypeStruct((B, M, DIM), F32),
            jax.ShapeDtypeStruct((B, M, DIM), F32),
            jax.ShapeDtypeStruct((B, N, DIM), F32),
        ],
    )(sft, feat, Wls, b2(bls), Wk, b2(bk), Wv, b2(bv), Wq, b2(bq))

    idx, statsp = pl.pallas_call(
        _topk_body,
        grid=(B, NT),
        in_specs=[
            pl.BlockSpec((1, TN, 3), lambda b, t: (b, t, 0)),
            pl.BlockSpec((1, M, 3), lambda b, t: (b, 0, 0)),
            pl.BlockSpec(Wp1.shape, lambda b, t: (0, 0)),
            pl.BlockSpec((1, POS_H), lambda b, t: (0, 0)),
        ],
        out_specs=[
            pl.BlockSpec((1, TN, K), lambda b, t: (b, t, 0)),
            pl.BlockSpec((2, POS_H), lambda b, t: (0, 0)),
        ],
        out_shape=[
            jax.ShapeDtypeStruct((B, N, K), jnp.int32),
            jax.ShapeDtypeStruct((2, POS_H), F32),
        ],
    )(pos_flipped, seed, Wp1, b2(bp1))

    ta, vpe, statsa = pl.pallas_call(
        _pass2_body,
        grid=(B, NT),
        in_specs=[
            pl.BlockSpec((1, TN, K), lambda b, t: (b, t, 0)),
            pl.BlockSpec((1, TN, 3), lambda b, t: (b, t, 0)),
            pl.BlockSpec((1, M, 3), lambda b, t: (b, 0, 0)),
            pl.BlockSpec((1, M, DIM), lambda b, t: (b, 0, 0)),
            pl.BlockSpec((1, M, DIM), lambda b, t: (b, 0, 0)),
            pl.BlockSpec((1, TN, DIM), lambda b, t: (b, t, 0)),
            pl.BlockSpec((2, POS_H), lambda b, t: (0, 0)),
        ] + [pl.BlockSpec(w.shape, lambda b, t: tuple(0 for _ in w.shape))
             for w in (Wp1, b2(bp1), b2(gp1), b2(betap1), Wp2, b2(bp2),
                       Wa1, b2(ba1))],
        out_specs=[
            pl.BlockSpec((1, TNK, ATTN_H), lambda b, t: (b, t, 0)),
            pl.BlockSpec((1, TNK, DIM), lambda b, t: (b, t, 0)),
            pl.BlockSpec((2, ATTN_H), lambda b, t: (0, 0)),
        ],
        out_shape=[
            jax.ShapeDtypeStruct((B, N * K, ATTN_H), F32),
            jax.ShapeDtypeStruct((B, N * K, DIM), F32),
            jax.ShapeDtypeStruct((2, ATTN_H), F32),
        ],
    )(idx, pos_flipped, seed, keyf, val, query, statsp,
      Wp1, b2(bp1), b2(gp1), b2(betap1), Wp2, b2(bp2), Wa1, b2(ba1))

    out = pl.pallas_call(
        _pass3_body,
        grid=(B, NT),
        in_specs=[
            pl.BlockSpec((1, TNK, ATTN_H), lambda b, t: (b, t, 0)),
            pl.BlockSpec((1, TNK, DIM), lambda b, t: (b, t, 0)),
            pl.BlockSpec((1, TN, C_IN), lambda b, t: (b, t, 0)),
            pl.BlockSpec((2, ATTN_H), lambda b, t: (0, 0)),
        ] + [pl.BlockSpec(w.shape, lambda b, t: tuple(0 for _ in w.shape))
             for w in (b2(ga1), b2(betaa1), Wa2, b2(ba2), Wle, b2(ble))],
        out_specs=pl.BlockSpec((1, TN, C_IN), lambda b, t: (b, t, 0)),
        out_shape=jax.ShapeDtypeStruct((B, N, C_IN), F32),
    )(ta, vpe, feat, statsa, b2(ga1), b2(betaa1), Wa2, b2(ba2), Wle, b2(ble))

    return jnp.transpose(out, (0, 2, 1))


# speedup vs baseline: 5.0952x; 2.4174x over previous
import functools

import jax
import jax.numpy as jnp
from jax import lax
from jax.experimental import pallas as pl
from jax.experimental.pallas import tpu as pltpu
from jax.experimental.pallas import tpu_sc as plsc

B, N, M, C_IN, DIM, K = 2, 2048, 512, 256, 256, 16
POS_H, ATTN_H = 64, 512
TN = 128
NT = N // TN
TNK = TN * K
CNT = float(B * N * K)
EPS = 1e-5
F32 = jnp.float32

NWORK = 32
RPW = B * N * K // NWORK
CH = 64
TW = 640


def _mmT(a, b):
    return jax.lax.dot_general(a, b, (((1,), (1,)), ((), ())),
                               preferred_element_type=F32)


def _mm(a, b):
    return jax.lax.dot_general(a, b, (((1,), (0,)), ((), ())),
                               preferred_element_type=F32)


def _proj_body(sft_ref, feat_ref, seed_ref, Wls_ref, bls_ref, Wk_ref, bk_ref,
               Wv_ref, bv_ref, Wq_ref, bq_ref,
               tab_ref, query_ref):
    v0 = _mmT(sft_ref[0], Wls_ref[...]) + bls_ref[...]
    keyf = _mmT(v0, Wk_ref[...]) + bk_ref[...]
    val = _mmT(v0, Wv_ref[...]) + bv_ref[...]
    tab_ref[0] = jnp.concatenate(
        [keyf, val, seed_ref[0], jnp.zeros((M, TW - 2 * DIM - 3), F32)],
        axis=1)
    query_ref[0] = _mmT(feat_ref[0], Wq_ref[...]) + bq_ref[...]


def _topk_body(posf_ref, seed_ref, idx_ref):
    q = posf_ref[0]
    s = seed_ref[0]
    qn = jnp.sum(q * q, axis=1, keepdims=True)
    sn = jnp.sum(s * s, axis=1)
    d = jax.lax.dot_general(q, s, (((1,), (1,)), ((), ())),
                            preferred_element_type=F32)
    d = -2.0 * d
    d = d + qn
    d = d + sn[None, :]
    colid = jax.lax.broadcasted_iota(jnp.int32, (TN, M), 1)
    cols = []
    for _ in range(K):
        mn = jnp.min(d, axis=1, keepdims=True)
        am = jnp.min(jnp.where(d == mn, colid, M), axis=1)
        cols.append(am[:, None])
        d = jnp.where(colid == am[:, None], jnp.inf, d)
    idx_t = jnp.concatenate(cols, axis=1)
    idx_ref[0] = idx_t + pl.program_id(0) * M


def _gather_body(tab_hbm, idx_hbm, gtab_hbm, idx_v, rows_v, sem):
    wid = lax.axis_index("s") * 2 + lax.axis_index("c")
    base = wid * RPW
    pltpu.sync_copy(idx_hbm.at[pl.ds(base, RPW)], idx_v)

    def body(i, carry):
        off = i * CH
        ii = idx_v.at[pl.ds(off, CH)]
        pltpu.async_copy(tab_hbm.at[ii], rows_v, sem).wait()
        pltpu.sync_copy(rows_v, gtab_hbm.at[pl.ds(base + off, CH)])
        return carry

    lax.fori_loop(0, RPW // CH, body, 0)


def _stats1_body(gcrd_ref, posf_ref, Wp1_ref, bp1_ref, statsp_ref):
    cg = gcrd_ref[0][:, 0:3]
    q = posf_ref[0]
    qrep = jnp.broadcast_to(q[:, None, :], (TN, K, 3)).reshape(TNK, 3)
    pos_rel = qrep - cg
    dis = jnp.sqrt(jnp.sum(pos_rel * pos_rel, axis=1, keepdims=True))
    h = jnp.concatenate([dis, pos_rel], axis=1)
    tp = _mmT(h, Wp1_ref[...]) + bp1_ref[...]

    statsp_ref[0, 0, :] = jnp.sum(tp, axis=0)
    statsp_ref[0, 1, :] = jnp.sum(tp * tp, axis=0)


def _pass2_body(gtab_ref, posf_ref, query_ref,
                statsp_ref, Wp1_ref, bp1_ref, gp1_ref, betap1_ref,
                Wp2_ref, bp2_ref, Wa1_ref, ba1_ref,
                ta_ref, vpe_ref, statsa_ref):
    cg = gtab_ref[0][:, 2 * DIM:2 * DIM + 3]
    q = posf_ref[0]
    qrep = jnp.broadcast_to(q[:, None, :], (TN, K, 3)).reshape(TNK, 3)
    pos_rel = qrep - cg
    dis = jnp.sqrt(jnp.sum(pos_rel * pos_rel, axis=1, keepdims=True))
    h = jnp.concatenate([dis, pos_rel], axis=1)
    tp = _mmT(h, Wp1_ref[...]) + bp1_ref[...]

    sp = statsp_ref[...]
    mp = (jnp.sum(sp[:, 0, :], axis=0) / CNT)[None, :]
    vp = (jnp.sum(sp[:, 1, :], axis=0) / CNT)[None, :] - mp * mp
    xn = (tp - mp) / jnp.sqrt(vp + EPS) * gp1_ref[...] + betap1_ref[...]
    pe = _mmT(jnp.maximum(xn, 0.0), Wp2_ref[...]) + bp2_ref[...]

    key_g = gtab_ref[0][:, 0:DIM]
    val_g = gtab_ref[0][:, DIM:2 * DIM]
    qf = query_ref[0]
    qfrep = jnp.broadcast_to(qf[:, None, :], (TN, K, DIM)).reshape(TNK, DIM)
    x = qfrep - key_g + pe
    ta = _mmT(x, Wa1_ref[...]) + ba1_ref[...]
    ta_ref[0] = ta
    vpe_ref[0] = val_g + pe

    statsa_ref[0, 0, :] = jnp.sum(ta, axis=0)
    statsa_ref[0, 1, :] = jnp.sum(ta * ta, axis=0)


def _pass3_body(ta_ref, vpe_ref, feat_ref, statsa_ref,
                ga1_ref, betaa1_ref, Wa2_ref, ba2_ref, Wle_ref, ble_ref,
                out_ref):
    ta = ta_ref[0]
    sa = statsa_ref[...]
    ma = (jnp.sum(sa[:, 0, :], axis=0) / CNT)[None, :]
    va = (jnp.sum(sa[:, 1, :], axis=0) / CNT)[None, :] - ma * ma
    u = (ta - ma) / jnp.sqrt(va + EPS) * ga1_ref[...] + betaa1_ref[...]
    u = jnp.maximum(u, 0.0)
    w = _mmT(u, Wa2_ref[...]) + ba2_ref[...]
    w3 = w.reshape(TN, K, DIM)
    wmax = jnp.max(w3, axis=1, keepdims=True)
    e = jnp.exp(w3 - wmax)
    sm = e / jnp.sum(e, axis=1, keepdims=True)
    agg = jnp.sum(sm * vpe_ref[0].reshape(TN, K, DIM), axis=1)
    out_ref[0] = _mmT(agg, Wle_ref[...]) + ble_ref[...] + feat_ref[0]


def kernel(pos, pos_flipped, fea, seed, seed_fea, Wq, bq, Wk, bk, Wv, bv,
           Wp1, bp1, gp1, betap1, Wp2, bp2, Wa1, ba1, ga1, betaa1, Wa2, ba2,
           Wls, bls, Wle, ble):
    del pos
    feat = jnp.transpose(fea, (0, 2, 1))
    sft = jnp.transpose(seed_fea, (0, 2, 1))
    b2 = lambda v: v.reshape(1, -1)

    tab, query = pl.pallas_call(
        _proj_body,
        grid=(B,),
        in_specs=[
            pl.BlockSpec((1, M, C_IN), lambda b: (b, 0, 0)),
            pl.BlockSpec((1, N, C_IN), lambda b: (b, 0, 0)),
            pl.BlockSpec((1, M, 3), lambda b: (b, 0, 0)),
        ] + [pl.BlockSpec(w.shape, lambda b: tuple(0 for _ in w.shape))
             for w in (Wls, b2(bls), Wk, b2(bk), Wv, b2(bv), Wq, b2(bq))],
        out_specs=[
            pl.BlockSpec((1, M, TW), lambda b: (b, 0, 0)),
            pl.BlockSpec((1, N, DIM), lambda b: (b, 0, 0)),
        ],
        out_shape=[
            jax.ShapeDtypeStruct((B, M, TW), F32),
            jax.ShapeDtypeStruct((B, N, DIM), F32),
        ],
    )(sft, feat, seed, Wls, b2(bls), Wk, b2(bk), Wv, b2(bv), Wq, b2(bq))

    idx = pl.pallas_call(
        _topk_body,
        grid=(B, NT),
        in_specs=[
            pl.BlockSpec((1, TN, 3), lambda b, t: (b, t, 0)),
            pl.BlockSpec((1, M, 3), lambda b, t: (b, 0, 0)),
        ],
        out_specs=pl.BlockSpec((1, TN, K), lambda b, t: (b, t, 0)),
        out_shape=jax.ShapeDtypeStruct((B, N, K), jnp.int32),
    )(pos_flipped, seed)

    gather = functools.partial(
        pl.kernel,
        mesh=plsc.VectorSubcoreMesh(core_axis_name="c", subcore_axis_name="s"),
        out_type=jax.ShapeDtypeStruct((B * N * K, TW), F32),
        scratch_types=[
            pltpu.VMEM((RPW,), jnp.int32),
            pltpu.VMEM((CH, TW), F32),
            pltpu.SemaphoreType.DMA,
        ],
    )(_gather_body)
    gtab = gather(tab.reshape(B * M, TW), idx.reshape(B * N * K))
    gtab = gtab.reshape(B, N * K, TW)

    statsp = pl.pallas_call(
        _stats1_body,
        grid=(B, NT),
        in_specs=[
            pl.BlockSpec((1, TNK, 128), lambda b, t: (b, t, 4)),
            pl.BlockSpec((1, TN, 3), lambda b, t: (b, t, 0)),
            pl.BlockSpec(Wp1.shape, lambda b, t: (0, 0)),
            pl.BlockSpec((1, POS_H), lambda b, t: (0, 0)),
        ],
        out_specs=pl.BlockSpec((1, 2, POS_H), lambda b, t: (b * NT + t, 0, 0)),
        out_shape=jax.ShapeDtypeStruct((B * NT, 2, POS_H), F32),
    )(gtab, pos_flipped, Wp1, b2(bp1))

    ta, vpe, statsa = pl.pallas_call(
        _pass2_body,
        grid=(B, NT),
        in_specs=[
            pl.BlockSpec((1, TNK, TW), lambda b, t: (b, t, 0)),
            pl.BlockSpec((1, TN, 3), lambda b, t: (b, t, 0)),
            pl.BlockSpec((1, TN, DIM), lambda b, t: (b, t, 0)),
            pl.BlockSpec((B * NT, 2, POS_H), lambda b, t: (0, 0, 0)),
        ] + [pl.BlockSpec(w.shape, lambda b, t: tuple(0 for _ in w.shape))
             for w in (Wp1, b2(bp1), b2(gp1), b2(betap1), Wp2, b2(bp2),
                       Wa1, b2(ba1))],
        out_specs=[
            pl.BlockSpec((1, TNK, ATTN_H), lambda b, t: (b, t, 0)),
            pl.BlockSpec((1, TNK, DIM), lambda b, t: (b, t, 0)),
            pl.BlockSpec((1, 2, ATTN_H), lambda b, t: (b * NT + t, 0, 0)),
        ],
        out_shape=[
            jax.ShapeDtypeStruct((B, N * K, ATTN_H), F32),
            jax.ShapeDtypeStruct((B, N * K, DIM), F32),
            jax.ShapeDtypeStruct((B * NT, 2, ATTN_H), F32),
        ],
    )(gtab, pos_flipped, query, statsp,
      Wp1, b2(bp1), b2(gp1), b2(betap1), Wp2, b2(bp2), Wa1, b2(ba1))

    out = pl.pallas_call(
        _pass3_body,
        grid=(B, NT),
        in_specs=[
            pl.BlockSpec((1, TNK, ATTN_H), lambda b, t: (b, t, 0)),
            pl.BlockSpec((1, TNK, DIM), lambda b, t: (b, t, 0)),
            pl.BlockSpec((1, TN, C_IN), lambda b, t: (b, t, 0)),
            pl.BlockSpec((B * NT, 2, ATTN_H), lambda b, t: (0, 0, 0)),
        ] + [pl.BlockSpec(w.shape, lambda b, t: tuple(0 for _ in w.shape))
             for w in (b2(ga1), b2(betaa1), Wa2, b2(ba2), Wle, b2(ble))],
        out_specs=pl.BlockSpec((1, TN, C_IN), lambda b, t: (b, t, 0)),
        out_shape=jax.ShapeDtypeStruct((B, N, C_IN), F32),
    )(ta, vpe, feat, statsa, b2(ga1), b2(betaa1), Wa2, b2(ba2), Wle, b2(ble))

    return jnp.transpose(out, (0, 2, 1))
